# baseline (speedup 1.0000x reference)
import functools

import jax
import jax.numpy as jnp
from jax import lax
from jax.experimental import pallas as pl
from jax.experimental.pallas import tpu as pltpu
from jax.experimental.pallas import tpu_sc as plsc

N = 10000
E = 160000
D_IN = 256
H_FEATS = 512
N_CLASSES = 64
K = 10
ALPHA = 0.1

NC = 2
NS = 16
LANES = 16
NPAD = 10240
EPT = 5120
EPAD = NC * NS * EPT
CHUNK = 128
NCHUNK = EPT // CHUNK
NBUF = 4
NROUND = NCHUNK // NBUF
ROWS_PER_TILE = NPAD // NS
_MESH = plsc.VectorSubcoreMesh(core_axis_name="c", subcore_axis_name="s")
_SC_PARAMS = pltpu.CompilerParams(use_tc_tiling_on_sc=False)
_SC_PARAMS_NOLAYOUT = pltpu.CompilerParams(use_tc_tiling_on_sc=False,
                                           needs_layout_passes=False)

_BN = 1000


def _mlp_body(x_ref, w1_ref, b1_ref, w2_ref, b2_ref, o_ref):
    h = jnp.dot(x_ref[...], w1_ref[...], preferred_element_type=jnp.float32)
    h = jnp.maximum(h + b1_ref[...], 0.0)
    o_ref[...] = jnp.dot(h, w2_ref[...], preferred_element_type=jnp.float32) + b2_ref[...]


def _mlp(x, w1, b1, w2, b2):
    return pl.pallas_call(
        _mlp_body,
        grid=(N // _BN,),
        in_specs=[
            pl.BlockSpec((_BN, D_IN), lambda i: (i, 0)),
            pl.BlockSpec((D_IN, H_FEATS), lambda i: (0, 0)),
            pl.BlockSpec((1, H_FEATS), lambda i: (0, 0)),
            pl.BlockSpec((H_FEATS, N_CLASSES), lambda i: (0, 0)),
            pl.BlockSpec((1, N_CLASSES), lambda i: (0, 0)),
        ],
        out_specs=pl.BlockSpec((_BN, N_CLASSES), lambda i: (i, 0)),
        out_shape=jax.ShapeDtypeStruct((N, N_CLASSES), jnp.float32),
    )(x, w1, b1, w2, b2)


@functools.partial(
    pl.kernel,
    out_type=jax.ShapeDtypeStruct((NC, 2, NPAD), jnp.float32),
    mesh=_MESH,
    compiler_params=_SC_PARAMS_NOLAYOUT,
    scratch_types=[
        pltpu.VMEM((NCHUNK, CHUNK), jnp.int32),
        pltpu.VMEM((NCHUNK, CHUNK), jnp.int32),
        pltpu.VMEM((NPAD,), jnp.float32),
        pltpu.VMEM((NPAD,), jnp.float32),
        pltpu.VMEM((NS, ROWS_PER_TILE), jnp.float32),
        pltpu.VMEM((ROWS_PER_TILE,), jnp.float32),
        pltpu.VMEM_SHARED((NS, NPAD), jnp.float32),
        pltpu.VMEM_SHARED((NS, NPAD), jnp.float32),
        pltpu.SemaphoreType.DMA,
        pltpu.SemaphoreType.DMA,
    ],
)
def _sc_degrees(src_hbm, dst_hbm, out_hbm, sidx_v, didx_v, hsrc_v, hdst_v,
                acc_v, sum_v, stage_src_sh, stage_dst_sh, sem_i, sem_a):
    cid = lax.axis_index("c")
    sid = lax.axis_index("s")
    wid = cid * NS + sid

    pltpu.async_copy(src_hbm.at[wid], sidx_v, sem_i)
    pltpu.async_copy(dst_hbm.at[wid], didx_v, sem_i)

    @pl.loop(0, NPAD, step=LANES)
    def _(i):
        hsrc_v[pl.ds(i, LANES)] = jnp.zeros((LANES,), jnp.float32)
        hdst_v[pl.ds(i, LANES)] = jnp.zeros((LANES,), jnp.float32)

    pltpu.make_async_copy(src_hbm.at[wid], sidx_v, sem_i).wait()
    pltpu.make_async_copy(dst_hbm.at[wid], didx_v, sem_i).wait()

    ones16 = jnp.full((LANES,), 1.0, jnp.float32)

    @pl.loop(0, NCHUNK)
    def _(c):
        @pl.loop(0, CHUNK, step=LANES)
        def _(j):
            plsc.addupdate_scatter(hsrc_v, [sidx_v[c, pl.ds(j, LANES)]], ones16)
            plsc.addupdate_scatter(hdst_v, [didx_v[c, pl.ds(j, LANES)]], ones16)

    pltpu.sync_copy(hsrc_v, stage_src_sh.at[sid])
    pltpu.sync_copy(hdst_v, stage_dst_sh.at[sid])
    plsc.subcore_barrier()

    base = sid * ROWS_PER_TILE
    for which, stage in ((0, stage_src_sh), (1, stage_dst_sh)):
        for t in range(NS):
            pltpu.async_copy(stage.at[t, pl.ds(base, ROWS_PER_TILE)], acc_v.at[t], sem_a)
        for t in range(NS):
            pltpu.make_async_copy(stage.at[t, pl.ds(base, ROWS_PER_TILE)], acc_v.at[t],
                                  sem_a).wait()

        @pl.loop(0, ROWS_PER_TILE, step=LANES)
        def _(j):
            s = acc_v[0, pl.ds(j, LANES)]
            for t in range(1, NS):
                s = s + acc_v[t, pl.ds(j, LANES)]
            sum_v[pl.ds(j, LANES)] = s

        pltpu.sync_copy(sum_v, out_hbm.at[cid, which, pl.ds(base, ROWS_PER_TILE)])


@functools.partial(
    pl.kernel,
    out_type=jax.ShapeDtypeStruct((NC, NPAD, N_CLASSES), jnp.float32),
    mesh=_MESH,
    compiler_params=_SC_PARAMS,
    scratch_types=[
        pltpu.VMEM((NCHUNK, CHUNK), jnp.int32),
        pltpu.VMEM((NCHUNK, CHUNK), jnp.int32),
        pltpu.VMEM((NBUF, CHUNK, N_CLASSES), jnp.float32),
        pltpu.VMEM((CHUNK, N_CLASSES), jnp.float32),
        pltpu.VMEM_SHARED((NPAD, N_CLASSES), jnp.float32),
        pltpu.SemaphoreType.DMA((NBUF,)),
        pltpu.SemaphoreType.DMA((NBUF,)),
        pltpu.SemaphoreType.DMA,
    ],
)
def _sc_step(hs_hbm, src_hbm, dst_hbm, out_hbm, sidx_v, didx_v, rows_v, zero_v,
             agg_sh, gsem, ssem, sem_z):
    cid = lax.axis_index("c")
    sid = lax.axis_index("s")
    wid = cid * NS + sid

    pltpu.async_copy(src_hbm.at[wid], sidx_v, sem_z)
    pltpu.async_copy(dst_hbm.at[wid], didx_v, sem_z)

    @pl.loop(0, CHUNK)
    def _(r):
        @pl.loop(0, N_CLASSES, step=LANES)
        def _(j):
            zero_v[r, pl.ds(j, LANES)] = jnp.zeros((LANES,), jnp.float32)

    base = sid * ROWS_PER_TILE

    @pl.loop(0, ROWS_PER_TILE // CHUNK)
    def _(b):
        pltpu.sync_copy(zero_v, agg_sh.at[pl.ds(base + b * CHUNK, CHUNK)])

    pltpu.make_async_copy(src_hbm.at[wid], sidx_v, sem_z).wait()
    pltpu.make_async_copy(dst_hbm.at[wid], didx_v, sem_z).wait()
    plsc.subcore_barrier()

    def _gather(c, b):
        pltpu.async_copy(hs_hbm.at[sidx_v.at[c]], rows_v.at[b], gsem.at[b])

    def _gather_wait(c, b):
        pltpu.make_async_copy(hs_hbm.at[sidx_v.at[c]], rows_v.at[b], gsem.at[b]).wait()

    def _scatter(c, b):
        pltpu.async_copy(rows_v.at[b], agg_sh.at[didx_v.at[c]], ssem.at[b], add=True)

    def _scatter_wait(c, b):
        pltpu.make_async_copy(rows_v.at[b], agg_sh.at[didx_v.at[c]], ssem.at[b]).wait()

    for b in range(NBUF):
        _gather(b, b)

    @pl.loop(0, NROUND - 1)
    def _(r):
        c0 = r * NBUF
        for b in range(NBUF):
            _gather_wait(c0 + b, b)
            _scatter(c0 + b, b)
        for b in range(NBUF):
            _scatter_wait(c0 + b, b)
            _gather(c0 + NBUF + b, b)

    c0 = (NROUND - 1) * NBUF
    for b in range(NBUF):
        _gather_wait(c0 + b, b)
        _scatter(c0 + b, b)
    for b in range(NBUF):
        _scatter_wait(c0 + b, b)

    plsc.subcore_barrier()
    pltpu.sync_copy(agg_sh.at[pl.ds(base, ROWS_PER_TILE)],
                    out_hbm.at[cid, pl.ds(base, ROWS_PER_TILE)])


def _norm_body(deg_ref, h0_ref, sn_ref, dn_ref, hs_ref):
    d = deg_ref[...]
    dsrc = d[0, 0, :, 0:1] + d[1, 0, :, 0:1]
    ddst = d[0, 1, :, 0:1] + d[1, 1, :, 0:1]
    sn = lax.rsqrt(jnp.maximum(dsrc, 1.0))
    dn = lax.rsqrt(jnp.maximum(ddst, 1.0))
    sn_ref[...] = sn
    dn_ref[...] = dn
    hs_ref[...] = h0_ref[...] * sn


def _norm(degs, h0):
    return pl.pallas_call(
        _norm_body,
        grid=(N // _BN,),
        in_specs=[
            pl.BlockSpec((NC, 2, _BN, 1), lambda i: (0, 0, i, 0)),
            pl.BlockSpec((_BN, N_CLASSES), lambda i: (i, 0)),
        ],
        out_specs=[
            pl.BlockSpec((_BN, 1), lambda i: (i, 0)),
            pl.BlockSpec((_BN, 1), lambda i: (i, 0)),
            pl.BlockSpec((_BN, N_CLASSES), lambda i: (i, 0)),
        ],
        out_shape=[
            jax.ShapeDtypeStruct((N, 1), jnp.float32),
            jax.ShapeDtypeStruct((N, 1), jnp.float32),
            jax.ShapeDtypeStruct((N, N_CLASSES), jnp.float32),
        ],
    )(degs, h0)


def _blend_body(agg_ref, dn_ref, sn_ref, h0_ref, h_ref, hs_ref):
    a = agg_ref[...]
    h = (1.0 - ALPHA) * (a[0] + a[1]) * dn_ref[...] + ALPHA * h0_ref[...]
    h_ref[...] = h
    hs_ref[...] = h * sn_ref[...]


def _blend(aggs, dn, sn, h0):
    return pl.pallas_call(
        _blend_body,
        grid=(N // _BN,),
        in_specs=[
            pl.BlockSpec((NC, _BN, N_CLASSES), lambda i: (0, i, 0)),
            pl.BlockSpec((_BN, 1), lambda i: (i, 0)),
            pl.BlockSpec((_BN, 1), lambda i: (i, 0)),
            pl.BlockSpec((_BN, N_CLASSES), lambda i: (i, 0)),
        ],
        out_specs=[
            pl.BlockSpec((_BN, N_CLASSES), lambda i: (i, 0)),
            pl.BlockSpec((_BN, N_CLASSES), lambda i: (i, 0)),
        ],
        out_shape=[
            jax.ShapeDtypeStruct((N, N_CLASSES), jnp.float32),
            jax.ShapeDtypeStruct((N, N_CLASSES), jnp.float32),
        ],
    )(aggs, dn, sn, h0)


def kernel(features, edge_index, W1, b1, W2, b2):
    src = edge_index[0]
    dst = edge_index[1]
    pad = EPAD - E
    src_gath = jnp.concatenate([src, jnp.zeros((pad,), jnp.int32)])
    src_deg = jnp.concatenate([src, jnp.full((pad,), N, jnp.int32)])
    dst_pad = jnp.concatenate([dst, jnp.full((pad,), N, jnp.int32)])
    src_gath = src_gath.reshape(NC * NS, NCHUNK, CHUNK)
    src_deg = src_deg.reshape(NC * NS, NCHUNK, CHUNK)
    dst_pad = dst_pad.reshape(NC * NS, NCHUNK, CHUNK)

    h0 = _mlp(features, W1, b1.reshape(1, -1), W2, b2.reshape(1, -1))
    degs = _sc_degrees(src_deg, dst_pad)
    sn, dn, hs = _norm(degs.reshape(NC, 2, NPAD, 1), h0)
    def _step(carry, _):
        h, hs = carry
        aggs = _sc_step(hs, src_gath, dst_pad)
        h, hs = _blend(aggs, dn, sn, h0)
        return (h, hs), None

    (h, hs), _ = lax.scan(_step, (h0, hs), None, length=K)
    return h


# speedup vs baseline: 1.9969x; 1.9969x over previous
import functools

import jax
import jax.numpy as jnp
from jax import lax
from jax.experimental import pallas as pl
from jax.experimental.pallas import tpu as pltpu
from jax.experimental.pallas import tpu_sc as plsc

N = 10000
E = 160000
D_IN = 256
H_FEATS = 512
N_CLASSES = 64
K = 10
ALPHA = 0.1

NC = 2
NS = 16
LANES = 16
NPAD = 10240
EPT = 5120
EPAD = NC * NS * EPT
CHUNK = 128
NCHUNK = EPT // CHUNK
NBUF = 4
NROUND = NCHUNK // NBUF
ROWS_PER_TILE = NPAD // NS
_MESH = plsc.VectorSubcoreMesh(core_axis_name="c", subcore_axis_name="s")
_SC_PARAMS = pltpu.CompilerParams(use_tc_tiling_on_sc=False)
_SC_PARAMS_NOLAYOUT = pltpu.CompilerParams(use_tc_tiling_on_sc=False,
                                           needs_layout_passes=False)

_BN = 1000


def _mlp_body(x_ref, w1_ref, b1_ref, w2_ref, b2_ref, o_ref):
    h = jnp.dot(x_ref[...], w1_ref[...], preferred_element_type=jnp.float32)
    h = jnp.maximum(h + b1_ref[...], 0.0)
    o_ref[...] = jnp.dot(h, w2_ref[...], preferred_element_type=jnp.float32) + b2_ref[...]


def _mlp(x, w1, b1, w2, b2):
    return pl.pallas_call(
        _mlp_body,
        grid=(N // _BN,),
        in_specs=[
            pl.BlockSpec((_BN, D_IN), lambda i: (i, 0)),
            pl.BlockSpec((D_IN, H_FEATS), lambda i: (0, 0)),
            pl.BlockSpec((1, H_FEATS), lambda i: (0, 0)),
            pl.BlockSpec((H_FEATS, N_CLASSES), lambda i: (0, 0)),
            pl.BlockSpec((1, N_CLASSES), lambda i: (0, 0)),
        ],
        out_specs=pl.BlockSpec((_BN, N_CLASSES), lambda i: (i, 0)),
        out_shape=jax.ShapeDtypeStruct((N, N_CLASSES), jnp.float32),
    )(x, w1, b1, w2, b2)


@functools.partial(
    pl.kernel,
    out_type=jax.ShapeDtypeStruct((NC, 2, NPAD), jnp.float32),
    mesh=_MESH,
    compiler_params=_SC_PARAMS_NOLAYOUT,
    scratch_types=[
        pltpu.VMEM((NCHUNK, CHUNK), jnp.int32),
        pltpu.VMEM((NCHUNK, CHUNK), jnp.int32),
        pltpu.VMEM((NPAD,), jnp.float32),
        pltpu.VMEM((NPAD,), jnp.float32),
        pltpu.VMEM((NS, ROWS_PER_TILE), jnp.float32),
        pltpu.VMEM((ROWS_PER_TILE,), jnp.float32),
        pltpu.VMEM_SHARED((NS, NPAD), jnp.float32),
        pltpu.VMEM_SHARED((NS, NPAD), jnp.float32),
        pltpu.SemaphoreType.DMA,
        pltpu.SemaphoreType.DMA,
    ],
)
def _sc_degrees(src_hbm, dst_hbm, out_hbm, sidx_v, didx_v, hsrc_v, hdst_v,
                acc_v, sum_v, stage_src_sh, stage_dst_sh, sem_i, sem_a):
    cid = lax.axis_index("c")
    sid = lax.axis_index("s")
    wid = cid * NS + sid

    pltpu.async_copy(src_hbm.at[wid], sidx_v, sem_i)
    pltpu.async_copy(dst_hbm.at[wid], didx_v, sem_i)

    @pl.loop(0, NPAD, step=LANES)
    def _(i):
        hsrc_v[pl.ds(i, LANES)] = jnp.zeros((LANES,), jnp.float32)
        hdst_v[pl.ds(i, LANES)] = jnp.zeros((LANES,), jnp.float32)

    pltpu.make_async_copy(src_hbm.at[wid], sidx_v, sem_i).wait()
    pltpu.make_async_copy(dst_hbm.at[wid], didx_v, sem_i).wait()

    ones16 = jnp.full((LANES,), 1.0, jnp.float32)

    @pl.loop(0, NCHUNK)
    def _(c):
        @pl.loop(0, CHUNK, step=LANES)
        def _(j):
            plsc.addupdate_scatter(hsrc_v, [sidx_v[c, pl.ds(j, LANES)]], ones16)
            plsc.addupdate_scatter(hdst_v, [didx_v[c, pl.ds(j, LANES)]], ones16)

    pltpu.sync_copy(hsrc_v, stage_src_sh.at[sid])
    pltpu.sync_copy(hdst_v, stage_dst_sh.at[sid])
    plsc.subcore_barrier()

    base = sid * ROWS_PER_TILE
    for which, stage in ((0, stage_src_sh), (1, stage_dst_sh)):
        for t in range(NS):
            pltpu.async_copy(stage.at[t, pl.ds(base, ROWS_PER_TILE)], acc_v.at[t], sem_a)
        for t in range(NS):
            pltpu.make_async_copy(stage.at[t, pl.ds(base, ROWS_PER_TILE)], acc_v.at[t],
                                  sem_a).wait()

        @pl.loop(0, ROWS_PER_TILE, step=LANES)
        def _(j):
            s = acc_v[0, pl.ds(j, LANES)]
            for t in range(1, NS):
                s = s + acc_v[t, pl.ds(j, LANES)]
            sum_v[pl.ds(j, LANES)] = s

        pltpu.sync_copy(sum_v, out_hbm.at[cid, which, pl.ds(base, ROWS_PER_TILE)])


@functools.partial(
    pl.kernel,
    out_type=jax.ShapeDtypeStruct((NC, NPAD, N_CLASSES), jnp.float32),
    mesh=_MESH,
    compiler_params=_SC_PARAMS,
    scratch_types=[
        pltpu.VMEM((NCHUNK, CHUNK), jnp.int32),
        pltpu.VMEM((NCHUNK, CHUNK), jnp.int32),
        pltpu.VMEM((NBUF, CHUNK, N_CLASSES), jnp.float32),
        pltpu.VMEM_SHARED((NPAD, N_CLASSES), jnp.float32),
        pltpu.VMEM_SHARED((NPAD, N_CLASSES), jnp.float32),
        pltpu.SemaphoreType.DMA((NBUF,)),
        pltpu.SemaphoreType.DMA((NBUF,)),
        pltpu.SemaphoreType.DMA,
    ],
)
def _sc_step(hs_hbm, src_hbm, dst_hbm, out_hbm, sidx_v, didx_v, rows_v,
             hs_sh, agg_sh, gsem, ssem, sem_z):
    cid = lax.axis_index("c")
    sid = lax.axis_index("s")
    wid = cid * NS + sid

    pltpu.async_copy(src_hbm.at[wid], sidx_v, sem_z)
    pltpu.async_copy(dst_hbm.at[wid], didx_v, sem_z)
    hrows = N // NS
    pltpu.async_copy(hs_hbm.at[pl.ds(sid * hrows, hrows)],
                     hs_sh.at[pl.ds(sid * hrows, hrows)], sem_z)

    zero_v = rows_v.at[0]

    @pl.loop(0, CHUNK)
    def _(r):
        @pl.loop(0, N_CLASSES, step=LANES)
        def _(j):
            zero_v[r, pl.ds(j, LANES)] = jnp.zeros((LANES,), jnp.float32)

    base = sid * ROWS_PER_TILE

    @pl.loop(0, ROWS_PER_TILE // CHUNK)
    def _(b):
        pltpu.sync_copy(zero_v, agg_sh.at[pl.ds(base + b * CHUNK, CHUNK)])

    pltpu.make_async_copy(src_hbm.at[wid], sidx_v, sem_z).wait()
    pltpu.make_async_copy(dst_hbm.at[wid], didx_v, sem_z).wait()
    pltpu.make_async_copy(hs_hbm.at[pl.ds(sid * hrows, hrows)],
                          hs_sh.at[pl.ds(sid * hrows, hrows)], sem_z).wait()
    plsc.subcore_barrier()

    def _gather(c, b):
        pltpu.async_copy(hs_sh.at[sidx_v.at[c]], rows_v.at[b], gsem.at[b])

    def _gather_wait(c, b):
        pltpu.make_async_copy(hs_sh.at[sidx_v.at[c]], rows_v.at[b], gsem.at[b]).wait()

    def _scatter(c, b):
        pltpu.async_copy(rows_v.at[b], agg_sh.at[didx_v.at[c]], ssem.at[b], add=True)

    def _scatter_wait(c, b):
        pltpu.make_async_copy(rows_v.at[b], agg_sh.at[didx_v.at[c]], ssem.at[b]).wait()

    for b in range(NBUF):
        _gather(b, b)

    @pl.loop(0, NROUND - 1)
    def _(r):
        c0 = r * NBUF
        for b in range(NBUF):
            _gather_wait(c0 + b, b)
            _scatter(c0 + b, b)
        for b in range(NBUF):
            _scatter_wait(c0 + b, b)
            _gather(c0 + NBUF + b, b)

    c0 = (NROUND - 1) * NBUF
    for b in range(NBUF):
        _gather_wait(c0 + b, b)
        _scatter(c0 + b, b)
    for b in range(NBUF):
        _scatter_wait(c0 + b, b)

    plsc.subcore_barrier()
    pltpu.sync_copy(agg_sh.at[pl.ds(base, ROWS_PER_TILE)],
                    out_hbm.at[cid, pl.ds(base, ROWS_PER_TILE)])


def _norm_body(deg_ref, h0_ref, sn_ref, dn_ref, hs_ref):
    d = deg_ref[...]
    dsrc = d[0, 0, :, 0:1] + d[1, 0, :, 0:1]
    ddst = d[0, 1, :, 0:1] + d[1, 1, :, 0:1]
    sn = lax.rsqrt(jnp.maximum(dsrc, 1.0))
    dn = lax.rsqrt(jnp.maximum(ddst, 1.0))
    sn_ref[...] = sn
    dn_ref[...] = dn
    hs_ref[...] = h0_ref[...] * sn


def _norm(degs, h0):
    return pl.pallas_call(
        _norm_body,
        grid=(N // _BN,),
        in_specs=[
            pl.BlockSpec((NC, 2, _BN, 1), lambda i: (0, 0, i, 0)),
            pl.BlockSpec((_BN, N_CLASSES), lambda i: (i, 0)),
        ],
        out_specs=[
            pl.BlockSpec((_BN, 1), lambda i: (i, 0)),
            pl.BlockSpec((_BN, 1), lambda i: (i, 0)),
            pl.BlockSpec((_BN, N_CLASSES), lambda i: (i, 0)),
        ],
        out_shape=[
            jax.ShapeDtypeStruct((N, 1), jnp.float32),
            jax.ShapeDtypeStruct((N, 1), jnp.float32),
            jax.ShapeDtypeStruct((N, N_CLASSES), jnp.float32),
        ],
    )(degs, h0)


def _blend_body(agg_ref, dn_ref, sn_ref, h0_ref, h_ref, hs_ref):
    a = agg_ref[...]
    h = (1.0 - ALPHA) * (a[0] + a[1]) * dn_ref[...] + ALPHA * h0_ref[...]
    h_ref[...] = h
    hs_ref[...] = h * sn_ref[...]


def _blend(aggs, dn, sn, h0):
    return pl.pallas_call(
        _blend_body,
        grid=(N // _BN,),
        in_specs=[
            pl.BlockSpec((NC, _BN, N_CLASSES), lambda i: (0, i, 0)),
            pl.BlockSpec((_BN, 1), lambda i: (i, 0)),
            pl.BlockSpec((_BN, 1), lambda i: (i, 0)),
            pl.BlockSpec((_BN, N_CLASSES), lambda i: (i, 0)),
        ],
        out_specs=[
            pl.BlockSpec((_BN, N_CLASSES), lambda i: (i, 0)),
            pl.BlockSpec((_BN, N_CLASSES), lambda i: (i, 0)),
        ],
        out_shape=[
            jax.ShapeDtypeStruct((N, N_CLASSES), jnp.float32),
            jax.ShapeDtypeStruct((N, N_CLASSES), jnp.float32),
        ],
    )(aggs, dn, sn, h0)


def kernel(features, edge_index, W1, b1, W2, b2):
    src = edge_index[0]
    dst = edge_index[1]
    pad = EPAD - E
    src_gath = jnp.concatenate([src, jnp.zeros((pad,), jnp.int32)])
    src_deg = jnp.concatenate([src, jnp.full((pad,), N, jnp.int32)])
    dst_pad = jnp.concatenate([dst, jnp.full((pad,), N, jnp.int32)])
    src_gath = src_gath.reshape(NC * NS, NCHUNK, CHUNK)
    src_deg = src_deg.reshape(NC * NS, NCHUNK, CHUNK)
    dst_pad = dst_pad.reshape(NC * NS, NCHUNK, CHUNK)

    h0 = _mlp(features, W1, b1.reshape(1, -1), W2, b2.reshape(1, -1))
    degs = _sc_degrees(src_deg, dst_pad)
    sn, dn, hs = _norm(degs.reshape(NC, 2, NPAD, 1), h0)
    def _step(carry, _):
        h, hs = carry
        aggs = _sc_step(hs, src_gath, dst_pad)
        h, hs = _blend(aggs, dn, sn, h0)
        return (h, hs), None

    (h, hs), _ = lax.scan(_step, (h0, hs), None, length=K)
    return h


# speedup vs baseline: 2.3284x; 1.1660x over previous
import functools

import jax
import jax.numpy as jnp
from jax import lax
from jax.experimental import pallas as pl
from jax.experimental.pallas import tpu as pltpu
from jax.experimental.pallas import tpu_sc as plsc

N = 10000
E = 160000
D_IN = 256
H_FEATS = 512
N_CLASSES = 64
K = 10
ALPHA = 0.1

NC = 2
NS = 16
LANES = 16
NPAD = 10240
EPT = 5120
EPAD = NC * NS * EPT
CHUNK = 128
NCHUNK = EPT // CHUNK
NBUF = 4
NROUND = NCHUNK // NBUF
ROWS_PER_TILE = NPAD // NS
_MESH = plsc.VectorSubcoreMesh(core_axis_name="c", subcore_axis_name="s")
_SC_PARAMS = pltpu.CompilerParams(use_tc_tiling_on_sc=False)
_SC_PARAMS_NOLAYOUT = pltpu.CompilerParams(use_tc_tiling_on_sc=False,
                                           needs_layout_passes=False)

_BN = 1024
_BF = _BN * N_CLASSES


def _mlp_body(x_ref, w1_ref, b1_ref, w2_ref, b2_ref, o_ref):
    h = jnp.dot(x_ref[...], w1_ref[...], preferred_element_type=jnp.float32)
    h = jnp.maximum(h + b1_ref[...], 0.0)
    o_ref[...] = jnp.dot(h, w2_ref[...], preferred_element_type=jnp.float32) + b2_ref[...]


def _mlp(x, w1, b1, w2, b2):
    return pl.pallas_call(
        _mlp_body,
        grid=(NPAD // _BN,),
        in_specs=[
            pl.BlockSpec((_BN, D_IN), lambda i: (i, 0)),
            pl.BlockSpec((D_IN, H_FEATS), lambda i: (0, 0)),
            pl.BlockSpec((1, H_FEATS), lambda i: (0, 0)),
            pl.BlockSpec((H_FEATS, N_CLASSES), lambda i: (0, 0)),
            pl.BlockSpec((1, N_CLASSES), lambda i: (0, 0)),
        ],
        out_specs=pl.BlockSpec((_BN, N_CLASSES), lambda i: (i, 0)),
        out_shape=jax.ShapeDtypeStruct((NPAD, N_CLASSES), jnp.float32),
    )(x, w1, b1, w2, b2)


@functools.partial(
    pl.kernel,
    out_type=jax.ShapeDtypeStruct((NC, 2, NPAD), jnp.float32),
    mesh=_MESH,
    compiler_params=_SC_PARAMS_NOLAYOUT,
    scratch_types=[
        pltpu.VMEM((NCHUNK, CHUNK), jnp.int32),
        pltpu.VMEM((NCHUNK, CHUNK), jnp.int32),
        pltpu.VMEM((NPAD,), jnp.float32),
        pltpu.VMEM((NPAD,), jnp.float32),
        pltpu.VMEM((NS, ROWS_PER_TILE), jnp.float32),
        pltpu.VMEM((ROWS_PER_TILE,), jnp.float32),
        pltpu.VMEM_SHARED((NS, NPAD), jnp.float32),
        pltpu.VMEM_SHARED((NS, NPAD), jnp.float32),
        pltpu.SemaphoreType.DMA,
        pltpu.SemaphoreType.DMA,
    ],
)
def _sc_degrees(src_hbm, dst_hbm, out_hbm, sidx_v, didx_v, hsrc_v, hdst_v,
                acc_v, sum_v, stage_src_sh, stage_dst_sh, sem_i, sem_a):
    cid = lax.axis_index("c")
    sid = lax.axis_index("s")
    wid = cid * NS + sid

    pltpu.async_copy(src_hbm.at[wid], sidx_v, sem_i)
    pltpu.async_copy(dst_hbm.at[wid], didx_v, sem_i)

    @pl.loop(0, NPAD, step=LANES)
    def _(i):
        hsrc_v[pl.ds(i, LANES)] = jnp.zeros((LANES,), jnp.float32)
        hdst_v[pl.ds(i, LANES)] = jnp.zeros((LANES,), jnp.float32)

    pltpu.make_async_copy(src_hbm.at[wid], sidx_v, sem_i).wait()
    pltpu.make_async_copy(dst_hbm.at[wid], didx_v, sem_i).wait()

    ones16 = jnp.full((LANES,), 1.0, jnp.float32)

    @pl.loop(0, NCHUNK)
    def _(c):
        @pl.loop(0, CHUNK, step=LANES)
        def _(j):
            plsc.addupdate_scatter(hsrc_v, [sidx_v[c, pl.ds(j, LANES)]], ones16)
            plsc.addupdate_scatter(hdst_v, [didx_v[c, pl.ds(j, LANES)]], ones16)

    pltpu.sync_copy(hsrc_v, stage_src_sh.at[sid])
    pltpu.sync_copy(hdst_v, stage_dst_sh.at[sid])
    plsc.subcore_barrier()

    base = sid * ROWS_PER_TILE
    for which, stage in ((0, stage_src_sh), (1, stage_dst_sh)):
        for t in range(NS):
            pltpu.async_copy(stage.at[t, pl.ds(base, ROWS_PER_TILE)], acc_v.at[t], sem_a)
        for t in range(NS):
            pltpu.make_async_copy(stage.at[t, pl.ds(base, ROWS_PER_TILE)], acc_v.at[t],
                                  sem_a).wait()

        @pl.loop(0, ROWS_PER_TILE, step=LANES)
        def _(j):
            s = acc_v[0, pl.ds(j, LANES)]
            for t in range(1, NS):
                s = s + acc_v[t, pl.ds(j, LANES)]
            sum_v[pl.ds(j, LANES)] = s

        pltpu.sync_copy(sum_v, out_hbm.at[cid, which, pl.ds(base, ROWS_PER_TILE)])


@functools.partial(
    pl.kernel,
    out_type=jax.ShapeDtypeStruct((NC, NPAD, N_CLASSES), jnp.float32),
    mesh=_MESH,
    compiler_params=_SC_PARAMS,
    scratch_types=[
        pltpu.VMEM((NCHUNK, CHUNK), jnp.int32),
        pltpu.VMEM((NCHUNK, CHUNK), jnp.int32),
        pltpu.VMEM((NBUF, CHUNK, N_CLASSES), jnp.float32),
        pltpu.VMEM_SHARED((NPAD, N_CLASSES), jnp.float32),
        pltpu.VMEM_SHARED((NPAD, N_CLASSES), jnp.float32),
        pltpu.SemaphoreType.DMA((NBUF,)),
        pltpu.SemaphoreType.DMA((NBUF,)),
        pltpu.SemaphoreType.DMA,
    ],
)
def _sc_step(hs_hbm, src_hbm, dst_hbm, out_hbm, sidx_v, didx_v, rows_v,
             hs_sh, agg_sh, gsem, ssem, sem_z):
    cid = lax.axis_index("c")
    sid = lax.axis_index("s")
    wid = cid * NS + sid

    pltpu.async_copy(src_hbm.at[wid], sidx_v, sem_z)
    pltpu.async_copy(dst_hbm.at[wid], didx_v, sem_z)
    hrows = NPAD // NS
    pltpu.async_copy(hs_hbm.at[pl.ds(sid * hrows, hrows)],
                     hs_sh.at[pl.ds(sid * hrows, hrows)], sem_z)

    zero_v = rows_v.at[0]

    @pl.loop(0, CHUNK)
    def _(r):
        @pl.loop(0, N_CLASSES, step=LANES)
        def _(j):
            zero_v[r, pl.ds(j, LANES)] = jnp.zeros((LANES,), jnp.float32)

    base = sid * ROWS_PER_TILE

    @pl.loop(0, ROWS_PER_TILE // CHUNK)
    def _(b):
        pltpu.sync_copy(zero_v, agg_sh.at[pl.ds(base + b * CHUNK, CHUNK)])

    pltpu.make_async_copy(src_hbm.at[wid], sidx_v, sem_z).wait()
    pltpu.make_async_copy(dst_hbm.at[wid], didx_v, sem_z).wait()
    pltpu.make_async_copy(hs_hbm.at[pl.ds(sid * hrows, hrows)],
                          hs_sh.at[pl.ds(sid * hrows, hrows)], sem_z).wait()
    plsc.subcore_barrier()

    def _gather(c, b):
        pltpu.async_copy(hs_sh.at[sidx_v.at[c]], rows_v.at[b], gsem.at[b])

    def _gather_wait(c, b):
        pltpu.make_async_copy(hs_sh.at[sidx_v.at[c]], rows_v.at[b], gsem.at[b]).wait()

    def _scatter(c, b):
        pltpu.async_copy(rows_v.at[b], agg_sh.at[didx_v.at[c]], ssem.at[b], add=True)

    def _scatter_wait(c, b):
        pltpu.make_async_copy(rows_v.at[b], agg_sh.at[didx_v.at[c]], ssem.at[b]).wait()

    for b in range(NBUF):
        _gather(b, b)

    @pl.loop(0, NROUND - 1)
    def _(r):
        c0 = r * NBUF
        for b in range(NBUF):
            _gather_wait(c0 + b, b)
            _scatter(c0 + b, b)
        for b in range(NBUF):
            _scatter_wait(c0 + b, b)
            _gather(c0 + NBUF + b, b)

    c0 = (NROUND - 1) * NBUF
    for b in range(NBUF):
        _gather_wait(c0 + b, b)
        _scatter(c0 + b, b)
    for b in range(NBUF):
        _scatter_wait(c0 + b, b)

    plsc.subcore_barrier()
    pltpu.sync_copy(agg_sh.at[pl.ds(base, ROWS_PER_TILE)],
                    out_hbm.at[cid, pl.ds(base, ROWS_PER_TILE)])


def _norm_body(deg_ref, h0_ref, sn64_ref, dn64_ref, hs_ref):
    d = deg_ref[...]
    dsrc = d[0, 0, :, 0:1] + d[1, 0, :, 0:1]
    ddst = d[0, 1, :, 0:1] + d[1, 1, :, 0:1]
    sn = lax.rsqrt(jnp.maximum(dsrc, 1.0))
    dn = lax.rsqrt(jnp.maximum(ddst, 1.0))
    sn64_ref[...] = jnp.broadcast_to(sn, (_BN, N_CLASSES))
    dn64_ref[...] = jnp.broadcast_to(dn, (_BN, N_CLASSES))
    hs_ref[...] = h0_ref[...] * sn


def _norm(degs, h0):
    return pl.pallas_call(
        _norm_body,
        grid=(NPAD // _BN,),
        in_specs=[
            pl.BlockSpec((NC, 2, _BN, 1), lambda i: (0, 0, i, 0)),
            pl.BlockSpec((_BN, N_CLASSES), lambda i: (i, 0)),
        ],
        out_specs=[
            pl.BlockSpec((_BN, N_CLASSES), lambda i: (i, 0)),
            pl.BlockSpec((_BN, N_CLASSES), lambda i: (i, 0)),
            pl.BlockSpec((_BN, N_CLASSES), lambda i: (i, 0)),
        ],
        out_shape=[
            jax.ShapeDtypeStruct((NPAD, N_CLASSES), jnp.float32),
            jax.ShapeDtypeStruct((NPAD, N_CLASSES), jnp.float32),
            jax.ShapeDtypeStruct((NPAD, N_CLASSES), jnp.float32),
        ],
    )(degs, h0)


def _blend_body(a0_ref, a1_ref, dn_ref, sn_ref, h0_ref, h_ref, hs_ref):
    h = ((1.0 - ALPHA) * (a0_ref[...] + a1_ref[...]) * dn_ref[...]
         + ALPHA * h0_ref[...])
    h_ref[...] = h
    hs_ref[...] = h * sn_ref[...]


def _blend(aggs1d, dn1d, sn1d, h01d):
    nflat = NPAD * N_CLASSES
    return pl.pallas_call(
        _blend_body,
        grid=(nflat // _BF,),
        in_specs=[
            pl.BlockSpec((_BF,), lambda i: (i,)),
            pl.BlockSpec((_BF,), lambda i: (i + NPAD // _BN,)),
            pl.BlockSpec((_BF,), lambda i: (i,)),
            pl.BlockSpec((_BF,), lambda i: (i,)),
            pl.BlockSpec((_BF,), lambda i: (i,)),
        ],
        out_specs=[
            pl.BlockSpec((_BF,), lambda i: (i,)),
            pl.BlockSpec((_BF,), lambda i: (i,)),
        ],
        out_shape=[
            jax.ShapeDtypeStruct((nflat,), jnp.float32),
            jax.ShapeDtypeStruct((nflat,), jnp.float32),
        ],
    )(aggs1d, aggs1d, dn1d, sn1d, h01d)


def kernel(features, edge_index, W1, b1, W2, b2):
    src = edge_index[0]
    dst = edge_index[1]
    pad = EPAD - E
    src_gath = jnp.concatenate([src, jnp.zeros((pad,), jnp.int32)])
    src_deg = jnp.concatenate([src, jnp.full((pad,), N, jnp.int32)])
    dst_pad = jnp.concatenate([dst, jnp.full((pad,), N, jnp.int32)])
    src_gath = src_gath.reshape(NC * NS, NCHUNK, CHUNK)
    src_deg = src_deg.reshape(NC * NS, NCHUNK, CHUNK)
    dst_pad = dst_pad.reshape(NC * NS, NCHUNK, CHUNK)

    xpad = jnp.concatenate(
        [features, jnp.zeros((NPAD - N, D_IN), jnp.float32)])
    h0 = _mlp(xpad, W1, b1.reshape(1, -1), W2, b2.reshape(1, -1))
    degs = _sc_degrees(src_deg, dst_pad)
    sn64, dn64, hs = _norm(degs.reshape(NC, 2, NPAD, 1), h0)
    sn1d = sn64.reshape(-1)
    dn1d = dn64.reshape(-1)
    h01d = h0.reshape(-1)
    hs1d = hs.reshape(-1)

    def _step(carry, _):
        h1d, hs1d = carry
        aggs = _sc_step(hs1d.reshape(NPAD, N_CLASSES), src_gath, dst_pad)
        h1d, hs1d = _blend(aggs.reshape(-1), dn1d, sn1d, h01d)
        return (h1d, hs1d), None

    (h1d, _), _ = lax.scan(_step, (h01d, hs1d), None, length=K)
    return h1d.reshape(NPAD, N_CLASSES)[:N]


# speedup vs baseline: 2.3677x; 1.0169x over previous
import functools

import jax
import jax.numpy as jnp
from jax import lax
from jax.experimental import pallas as pl
from jax.experimental.pallas import tpu as pltpu
from jax.experimental.pallas import tpu_sc as plsc

N = 10000
E = 160000
D_IN = 256
H_FEATS = 512
N_CLASSES = 64
K = 10
ALPHA = 0.1

NC = 2
NS = 16
LANES = 16
NPAD = 10240
EPT = 5120
EPAD = NC * NS * EPT
CHUNK = 128
NCHUNK = EPT // CHUNK
NBUF = 4
NROUND = NCHUNK // NBUF
ROWS_PER_TILE = NPAD // NS
_MESH = plsc.VectorSubcoreMesh(core_axis_name="c", subcore_axis_name="s")
_SC_PARAMS = pltpu.CompilerParams(use_tc_tiling_on_sc=False)
_SC_PARAMS_NOLAYOUT = pltpu.CompilerParams(use_tc_tiling_on_sc=False,
                                           needs_layout_passes=False)

_BN = 1024
_BF = _BN * N_CLASSES


def _mlp_body(x_ref, w1_ref, b1_ref, w2_ref, b2_ref, o_ref):
    h = jnp.dot(x_ref[...], w1_ref[...], preferred_element_type=jnp.float32)
    h = jnp.maximum(h + b1_ref[...], 0.0)
    o_ref[...] = jnp.dot(h, w2_ref[...], preferred_element_type=jnp.float32) + b2_ref[...]


def _mlp(x, w1, b1, w2, b2):
    return pl.pallas_call(
        _mlp_body,
        grid=(NPAD // _BN,),
        in_specs=[
            pl.BlockSpec((_BN, D_IN), lambda i: (i, 0)),
            pl.BlockSpec((D_IN, H_FEATS), lambda i: (0, 0)),
            pl.BlockSpec((1, H_FEATS), lambda i: (0, 0)),
            pl.BlockSpec((H_FEATS, N_CLASSES), lambda i: (0, 0)),
            pl.BlockSpec((1, N_CLASSES), lambda i: (0, 0)),
        ],
        out_specs=pl.BlockSpec((_BN, N_CLASSES), lambda i: (i, 0)),
        out_shape=jax.ShapeDtypeStruct((NPAD, N_CLASSES), jnp.float32),
    )(x, w1, b1, w2, b2)


def _rsqrt16(x):
    i = plsc.bitcast(x, jnp.int32)
    i = jnp.full((LANES,), 0x5F3759DF, jnp.int32) - lax.shift_right_logical(i, 1)
    y = plsc.bitcast(i, jnp.float32)
    for _ in range(3):
        y = y * (1.5 - 0.5 * x * y * y)
    return y


@functools.partial(
    pl.kernel,
    out_type=jax.ShapeDtypeStruct((2, NPAD, N_CLASSES), jnp.float32),
    mesh=_MESH,
    compiler_params=_SC_PARAMS_NOLAYOUT,
    scratch_types=[
        pltpu.VMEM((NCHUNK, CHUNK), jnp.int32),
        pltpu.VMEM((NPAD,), jnp.float32),
        pltpu.VMEM((NPAD,), jnp.float32),
        pltpu.VMEM((NS, ROWS_PER_TILE), jnp.float32),
        pltpu.VMEM((ROWS_PER_TILE,), jnp.float32),
        pltpu.VMEM((ROWS_PER_TILE, N_CLASSES), jnp.float32),
        pltpu.VMEM_SHARED((NS, NPAD), jnp.float32),
        pltpu.VMEM_SHARED((NS, NPAD), jnp.float32),
        pltpu.SemaphoreType.DMA,
    ],
)
def _sc_degrees(src_hbm, dst_hbm, out_hbm, idx_v, hsrc_v, hdst_v,
                acc_v, nrm_v, exp_v, stage_src_sh, stage_dst_sh, sem_a):
    cid = lax.axis_index("c")
    sid = lax.axis_index("s")

    @pl.when(cid == 0)
    def _():
        @pl.loop(0, NPAD, step=LANES)
        def _(i):
            hsrc_v[pl.ds(i, LANES)] = jnp.zeros((LANES,), jnp.float32)
            hdst_v[pl.ds(i, LANES)] = jnp.zeros((LANES,), jnp.float32)

        ones16 = jnp.full((LANES,), 1.0, jnp.float32)

        for part in range(2):
            for which, ehbm, hist in ((0, src_hbm, hsrc_v), (1, dst_hbm, hdst_v)):
                pltpu.sync_copy(ehbm.at[part * NS + sid], idx_v)

                @pl.loop(0, NCHUNK)
                def _(c):
                    @pl.loop(0, CHUNK, step=LANES)
                    def _(j):
                        plsc.addupdate_scatter(hist, [idx_v[c, pl.ds(j, LANES)]],
                                               ones16)

        pltpu.sync_copy(hsrc_v, stage_src_sh.at[sid])
        pltpu.sync_copy(hdst_v, stage_dst_sh.at[sid])
        plsc.subcore_barrier()

        base = sid * ROWS_PER_TILE
        for which, stage in ((0, stage_src_sh), (1, stage_dst_sh)):
            for t in range(NS):
                pltpu.async_copy(stage.at[t, pl.ds(base, ROWS_PER_TILE)],
                                 acc_v.at[t], sem_a)
            for t in range(NS):
                pltpu.make_async_copy(stage.at[t, pl.ds(base, ROWS_PER_TILE)],
                                      acc_v.at[t], sem_a).wait()

            @pl.loop(0, ROWS_PER_TILE, step=LANES)
            def _(j):
                s = acc_v[0, pl.ds(j, LANES)]
                for t in range(1, NS):
                    s = s + acc_v[t, pl.ds(j, LANES)]
                nrm_v[pl.ds(j, LANES)] = _rsqrt16(jnp.maximum(s, 1.0))

            @pl.loop(0, ROWS_PER_TILE)
            def _(r):
                v = plsc.load_gather(nrm_v, [jnp.full((LANES,), r, jnp.int32)])
                @pl.loop(0, N_CLASSES, step=LANES)
                def _(jj):
                    exp_v[r, pl.ds(jj, LANES)] = v

            pltpu.sync_copy(exp_v, out_hbm.at[which, pl.ds(base, ROWS_PER_TILE)])


@functools.partial(
    pl.kernel,
    out_type=jax.ShapeDtypeStruct((NC, NPAD, N_CLASSES), jnp.float32),
    mesh=_MESH,
    compiler_params=_SC_PARAMS,
    scratch_types=[
        pltpu.VMEM((NCHUNK, CHUNK), jnp.int32),
        pltpu.VMEM((NCHUNK, CHUNK), jnp.int32),
        pltpu.VMEM((NBUF, CHUNK, N_CLASSES), jnp.float32),
        pltpu.VMEM_SHARED((NPAD, N_CLASSES), jnp.float32),
        pltpu.VMEM_SHARED((NPAD, N_CLASSES), jnp.float32),
        pltpu.SemaphoreType.DMA((NBUF,)),
        pltpu.SemaphoreType.DMA((NBUF,)),
        pltpu.SemaphoreType.DMA,
    ],
)
def _sc_step(hs_hbm, src_hbm, dst_hbm, out_hbm, sidx_v, didx_v, rows_v,
             hs_sh, agg_sh, gsem, ssem, sem_z):
    cid = lax.axis_index("c")
    sid = lax.axis_index("s")
    wid = cid * NS + sid

    pltpu.async_copy(src_hbm.at[wid], sidx_v, sem_z)
    pltpu.async_copy(dst_hbm.at[wid], didx_v, sem_z)
    hrows = NPAD // NS
    pltpu.async_copy(hs_hbm.at[pl.ds(sid * hrows, hrows)],
                     hs_sh.at[pl.ds(sid * hrows, hrows)], sem_z)

    zero_v = rows_v.at[0]

    @pl.loop(0, CHUNK)
    def _(r):
        @pl.loop(0, N_CLASSES, step=LANES)
        def _(j):
            zero_v[r, pl.ds(j, LANES)] = jnp.zeros((LANES,), jnp.float32)

    base = sid * ROWS_PER_TILE

    @pl.loop(0, ROWS_PER_TILE // CHUNK)
    def _(b):
        pltpu.sync_copy(zero_v, agg_sh.at[pl.ds(base + b * CHUNK, CHUNK)])

    pltpu.make_async_copy(src_hbm.at[wid], sidx_v, sem_z).wait()
    pltpu.make_async_copy(dst_hbm.at[wid], didx_v, sem_z).wait()
    pltpu.make_async_copy(hs_hbm.at[pl.ds(sid * hrows, hrows)],
                          hs_sh.at[pl.ds(sid * hrows, hrows)], sem_z).wait()
    plsc.subcore_barrier()

    def _gather(c, b):
        pltpu.async_copy(hs_sh.at[sidx_v.at[c]], rows_v.at[b], gsem.at[b])

    def _gather_wait(c, b):
        pltpu.make_async_copy(hs_sh.at[sidx_v.at[c]], rows_v.at[b], gsem.at[b]).wait()

    def _scatter(c, b):
        pltpu.async_copy(rows_v.at[b], agg_sh.at[didx_v.at[c]], ssem.at[b], add=True)

    def _scatter_wait(c, b):
        pltpu.make_async_copy(rows_v.at[b], agg_sh.at[didx_v.at[c]], ssem.at[b]).wait()

    for b in range(NBUF):
        _gather(b, b)

    @pl.loop(0, NROUND - 1)
    def _(r):
        c0 = r * NBUF
        for b in range(NBUF):
            _gather_wait(c0 + b, b)
            _scatter(c0 + b, b)
        for b in range(NBUF):
            _scatter_wait(c0 + b, b)
            _gather(c0 + NBUF + b, b)

    c0 = (NROUND - 1) * NBUF
    for b in range(NBUF):
        _gather_wait(c0 + b, b)
        _scatter(c0 + b, b)
    for b in range(NBUF):
        _scatter_wait(c0 + b, b)

    plsc.subcore_barrier()
    pltpu.sync_copy(agg_sh.at[pl.ds(base, ROWS_PER_TILE)],
                    out_hbm.at[cid, pl.ds(base, ROWS_PER_TILE)])


def _scale_body(a_ref, b_ref, o_ref):
    o_ref[...] = a_ref[...] * b_ref[...]


def _scale(a1d, b1d):
    nflat = NPAD * N_CLASSES
    return pl.pallas_call(
        _scale_body,
        grid=(nflat // _BF,),
        in_specs=[
            pl.BlockSpec((_BF,), lambda i: (i,)),
            pl.BlockSpec((_BF,), lambda i: (i,)),
        ],
        out_specs=pl.BlockSpec((_BF,), lambda i: (i,)),
        out_shape=jax.ShapeDtypeStruct((nflat,), jnp.float32),
    )(a1d, b1d)


def _blend_body(a0_ref, a1_ref, dn_ref, sn_ref, h0_ref, h_ref, hs_ref):
    h = ((1.0 - ALPHA) * (a0_ref[...] + a1_ref[...]) * dn_ref[...]
         + ALPHA * h0_ref[...])
    h_ref[...] = h
    hs_ref[...] = h * sn_ref[...]


def _blend(aggs1d, dn1d, sn1d, h01d):
    nflat = NPAD * N_CLASSES
    return pl.pallas_call(
        _blend_body,
        grid=(nflat // _BF,),
        in_specs=[
            pl.BlockSpec((_BF,), lambda i: (i,)),
            pl.BlockSpec((_BF,), lambda i: (i + NPAD // _BN,)),
            pl.BlockSpec((_BF,), lambda i: (i,)),
            pl.BlockSpec((_BF,), lambda i: (i,)),
            pl.BlockSpec((_BF,), lambda i: (i,)),
        ],
        out_specs=[
            pl.BlockSpec((_BF,), lambda i: (i,)),
            pl.BlockSpec((_BF,), lambda i: (i,)),
        ],
        out_shape=[
            jax.ShapeDtypeStruct((nflat,), jnp.float32),
            jax.ShapeDtypeStruct((nflat,), jnp.float32),
        ],
    )(aggs1d, aggs1d, dn1d, sn1d, h01d)


def kernel(features, edge_index, W1, b1, W2, b2):
    src = edge_index[0]
    dst = edge_index[1]
    pad = EPAD - E
    src_gath = jnp.concatenate([src, jnp.zeros((pad,), jnp.int32)])
    src_deg = jnp.concatenate([src, jnp.full((pad,), N, jnp.int32)])
    dst_pad = jnp.concatenate([dst, jnp.full((pad,), N, jnp.int32)])
    src_gath = src_gath.reshape(NC * NS, NCHUNK, CHUNK)
    src_deg = src_deg.reshape(NC * NS, NCHUNK, CHUNK)
    dst_pad = dst_pad.reshape(NC * NS, NCHUNK, CHUNK)

    xpad = jnp.concatenate(
        [features, jnp.zeros((NPAD - N, D_IN), jnp.float32)])
    h0 = _mlp(xpad, W1, b1.reshape(1, -1), W2, b2.reshape(1, -1))
    norms = _sc_degrees(src_deg, dst_pad)
    sn1d = norms[0].reshape(-1)
    dn1d = norms[1].reshape(-1)
    h01d = h0.reshape(-1)
    hs1d = _scale(h01d, sn1d)

    def _step(carry, _):
        h1d, hs1d = carry
        aggs = _sc_step(hs1d.reshape(NPAD, N_CLASSES), src_gath, dst_pad)
        h1d, hs1d = _blend(aggs.reshape(-1), dn1d, sn1d, h01d)
        return (h1d, hs1d), None

    (h1d, _), _ = lax.scan(_step, (h01d, hs1d), None, length=K)
    return h1d.reshape(NPAD, N_CLASSES)[:N]


# speedup vs baseline: 2.4584x; 1.0383x over previous
import functools

import jax
import jax.numpy as jnp
from jax import lax
from jax.experimental import pallas as pl
from jax.experimental.pallas import tpu as pltpu
from jax.experimental.pallas import tpu_sc as plsc

N = 10000
E = 160000
D_IN = 256
H_FEATS = 512
N_CLASSES = 64
K = 10
ALPHA = 0.1

NC = 2
NS = 16
LANES = 16
NPAD = 10240
EPT = 5120
EPAD = NC * NS * EPT
CHUNK = 128
NCHUNK = EPT // CHUNK
NBUF = 4
NROUND = NCHUNK // NBUF
ROWS_PER_TILE = NPAD // NS
_MESH = plsc.VectorSubcoreMesh(core_axis_name="c", subcore_axis_name="s")
_SC_PARAMS = pltpu.CompilerParams(use_tc_tiling_on_sc=False)
_SC_PARAMS_NOLAYOUT = pltpu.CompilerParams(use_tc_tiling_on_sc=False,
                                           needs_layout_passes=False)

_BN = 1024
_BF = 2 * _BN * N_CLASSES


def _mlp_body(x_ref, w1_ref, b1_ref, w2_ref, b2_ref, o_ref):
    h = jnp.dot(x_ref[...], w1_ref[...], preferred_element_type=jnp.float32)
    h = jnp.maximum(h + b1_ref[...], 0.0)
    o_ref[...] = jnp.dot(h, w2_ref[...], preferred_element_type=jnp.float32) + b2_ref[...]


def _mlp(x, w1, b1, w2, b2):
    return pl.pallas_call(
        _mlp_body,
        grid=(NPAD // _BN,),
        in_specs=[
            pl.BlockSpec((_BN, D_IN), lambda i: (i, 0)),
            pl.BlockSpec((D_IN, H_FEATS), lambda i: (0, 0)),
            pl.BlockSpec((1, H_FEATS), lambda i: (0, 0)),
            pl.BlockSpec((H_FEATS, N_CLASSES), lambda i: (0, 0)),
            pl.BlockSpec((1, N_CLASSES), lambda i: (0, 0)),
        ],
        out_specs=pl.BlockSpec((_BN, N_CLASSES), lambda i: (i, 0)),
        out_shape=jax.ShapeDtypeStruct((NPAD, N_CLASSES), jnp.float32),
    )(x, w1, b1, w2, b2)


def _rsqrt16(x):
    i = plsc.bitcast(x, jnp.int32)
    i = jnp.full((LANES,), 0x5F3759DF, jnp.int32) - lax.shift_right_logical(i, 1)
    y = plsc.bitcast(i, jnp.float32)
    for _ in range(3):
        y = y * (1.5 - 0.5 * x * y * y)
    return y


@functools.partial(
    pl.kernel,
    out_type=jax.ShapeDtypeStruct((2, NPAD, N_CLASSES), jnp.float32),
    mesh=_MESH,
    compiler_params=_SC_PARAMS_NOLAYOUT,
    scratch_types=[
        pltpu.VMEM((NCHUNK, CHUNK), jnp.int32),
        pltpu.VMEM((NPAD,), jnp.float32),
        pltpu.VMEM((NPAD,), jnp.float32),
        pltpu.VMEM((NS, ROWS_PER_TILE), jnp.float32),
        pltpu.VMEM((ROWS_PER_TILE,), jnp.float32),
        pltpu.VMEM((ROWS_PER_TILE, N_CLASSES), jnp.float32),
        pltpu.VMEM_SHARED((NS, NPAD), jnp.float32),
        pltpu.VMEM_SHARED((NS, NPAD), jnp.float32),
        pltpu.SemaphoreType.DMA,
    ],
)
def _sc_degrees(src_hbm, dst_hbm, out_hbm, idx_v, hsrc_v, hdst_v,
                acc_v, nrm_v, exp_v, stage_src_sh, stage_dst_sh, sem_a):
    cid = lax.axis_index("c")
    sid = lax.axis_index("s")

    @pl.when(cid == 0)
    def _():
        @pl.loop(0, NPAD, step=LANES)
        def _(i):
            hsrc_v[pl.ds(i, LANES)] = jnp.zeros((LANES,), jnp.float32)
            hdst_v[pl.ds(i, LANES)] = jnp.zeros((LANES,), jnp.float32)

        ones16 = jnp.full((LANES,), 1.0, jnp.float32)

        for part in range(2):
            for which, ehbm, hist in ((0, src_hbm, hsrc_v), (1, dst_hbm, hdst_v)):
                pltpu.sync_copy(ehbm.at[part * NS + sid], idx_v)

                @pl.loop(0, NCHUNK)
                def _(c):
                    @pl.loop(0, CHUNK, step=LANES)
                    def _(j):
                        plsc.addupdate_scatter(hist, [idx_v[c, pl.ds(j, LANES)]],
                                               ones16)

        pltpu.sync_copy(hsrc_v, stage_src_sh.at[sid])
        pltpu.sync_copy(hdst_v, stage_dst_sh.at[sid])
        plsc.subcore_barrier()

        base = sid * ROWS_PER_TILE
        for which, stage in ((0, stage_src_sh), (1, stage_dst_sh)):
            for t in range(NS):
                pltpu.async_copy(stage.at[t, pl.ds(base, ROWS_PER_TILE)],
                                 acc_v.at[t], sem_a)
            for t in range(NS):
                pltpu.make_async_copy(stage.at[t, pl.ds(base, ROWS_PER_TILE)],
                                      acc_v.at[t], sem_a).wait()

            @pl.loop(0, ROWS_PER_TILE, step=LANES)
            def _(j):
                s = acc_v[0, pl.ds(j, LANES)]
                for t in range(1, NS):
                    s = s + acc_v[t, pl.ds(j, LANES)]
                nrm_v[pl.ds(j, LANES)] = _rsqrt16(jnp.maximum(s, 1.0))

            @pl.loop(0, ROWS_PER_TILE)
            def _(r):
                v = plsc.load_gather(nrm_v, [jnp.full((LANES,), r, jnp.int32)])
                @pl.loop(0, N_CLASSES, step=LANES)
                def _(jj):
                    exp_v[r, pl.ds(jj, LANES)] = v

            pltpu.sync_copy(exp_v, out_hbm.at[which, pl.ds(base, ROWS_PER_TILE)])


@functools.partial(
    pl.kernel,
    out_type=jax.ShapeDtypeStruct((NC, NPAD, N_CLASSES), jnp.float32),
    mesh=_MESH,
    compiler_params=_SC_PARAMS,
    scratch_types=[
        pltpu.VMEM((NCHUNK, CHUNK), jnp.int32),
        pltpu.VMEM((NCHUNK, CHUNK), jnp.int32),
        pltpu.VMEM((NBUF, CHUNK, N_CLASSES), jnp.float32),
        pltpu.VMEM_SHARED((NPAD, N_CLASSES), jnp.float32),
        pltpu.VMEM_SHARED((NPAD, N_CLASSES), jnp.float32),
        pltpu.SemaphoreType.DMA((NBUF,)),
        pltpu.SemaphoreType.DMA((NBUF,)),
        pltpu.SemaphoreType.DMA,
    ],
)
def _sc_step(hs_hbm, src_hbm, dst_hbm, out_hbm, sidx_v, didx_v, rows_v,
             hs_sh, agg_sh, gsem, ssem, sem_z):
    cid = lax.axis_index("c")
    sid = lax.axis_index("s")
    wid = cid * NS + sid

    pltpu.async_copy(src_hbm.at[wid], sidx_v, sem_z)
    pltpu.async_copy(dst_hbm.at[wid], didx_v, sem_z)
    hrows = NPAD // NS
    pltpu.async_copy(hs_hbm.at[pl.ds(sid * hrows, hrows)],
                     hs_sh.at[pl.ds(sid * hrows, hrows)], sem_z)

    zero_v = rows_v.at[0]

    @pl.loop(0, CHUNK)
    def _(r):
        @pl.loop(0, N_CLASSES, step=LANES)
        def _(j):
            zero_v[r, pl.ds(j, LANES)] = jnp.zeros((LANES,), jnp.float32)

    base = sid * ROWS_PER_TILE

    @pl.loop(0, ROWS_PER_TILE // CHUNK)
    def _(b):
        pltpu.sync_copy(zero_v, agg_sh.at[pl.ds(base + b * CHUNK, CHUNK)])

    pltpu.make_async_copy(src_hbm.at[wid], sidx_v, sem_z).wait()
    pltpu.make_async_copy(dst_hbm.at[wid], didx_v, sem_z).wait()
    pltpu.make_async_copy(hs_hbm.at[pl.ds(sid * hrows, hrows)],
                          hs_sh.at[pl.ds(sid * hrows, hrows)], sem_z).wait()
    plsc.subcore_barrier()

    def _gather(c, b):
        pltpu.async_copy(hs_sh.at[sidx_v.at[c]], rows_v.at[b], gsem.at[b])

    def _gather_wait(c, b):
        pltpu.make_async_copy(hs_sh.at[sidx_v.at[c]], rows_v.at[b], gsem.at[b]).wait()

    def _scatter(c, b):
        pltpu.async_copy(rows_v.at[b], agg_sh.at[didx_v.at[c]], ssem.at[b], add=True)

    def _scatter_wait(c, b):
        pltpu.make_async_copy(rows_v.at[b], agg_sh.at[didx_v.at[c]], ssem.at[b]).wait()

    for b in range(NBUF):
        _gather(b, b)

    @pl.loop(0, NROUND - 1)
    def _(r):
        c0 = r * NBUF
        for b in range(NBUF):
            _gather_wait(c0 + b, b)
            _scatter(c0 + b, b)
        for b in range(NBUF):
            _scatter_wait(c0 + b, b)
            _gather(c0 + NBUF + b, b)

    c0 = (NROUND - 1) * NBUF
    for b in range(NBUF):
        _gather_wait(c0 + b, b)
        _scatter(c0 + b, b)
    for b in range(NBUF):
        _scatter_wait(c0 + b, b)

    plsc.subcore_barrier()
    pltpu.sync_copy(agg_sh.at[pl.ds(base, ROWS_PER_TILE)],
                    out_hbm.at[cid, pl.ds(base, ROWS_PER_TILE)])


def _scale_body(a_ref, b_ref, o_ref):
    o_ref[...] = a_ref[...] * b_ref[...]


def _scale(a1d, b1d):
    nflat = NPAD * N_CLASSES
    return pl.pallas_call(
        _scale_body,
        grid=(nflat // _BF,),
        in_specs=[
            pl.BlockSpec((_BF,), lambda i: (i,)),
            pl.BlockSpec((_BF,), lambda i: (i,)),
        ],
        out_specs=pl.BlockSpec((_BF,), lambda i: (i,)),
        out_shape=jax.ShapeDtypeStruct((nflat,), jnp.float32),
    )(a1d, b1d)


def _blend_body(a0_ref, a1_ref, dn_ref, sn_ref, h0_ref, h_ref, hs_ref):
    h = ((1.0 - ALPHA) * (a0_ref[...] + a1_ref[...]) * dn_ref[...]
         + ALPHA * h0_ref[...])
    h_ref[...] = h
    hs_ref[...] = h * sn_ref[...]


def _blend(aggs1d, dn1d, sn1d, h01d):
    nflat = NPAD * N_CLASSES
    return pl.pallas_call(
        _blend_body,
        grid=(nflat // _BF,),
        in_specs=[
            pl.BlockSpec((_BF,), lambda i: (i,)),
            pl.BlockSpec((_BF,), lambda i: (i + NPAD * N_CLASSES // _BF,)),
            pl.BlockSpec((_BF,), lambda i: (i,)),
            pl.BlockSpec((_BF,), lambda i: (i,)),
            pl.BlockSpec((_BF,), lambda i: (i,)),
        ],
        out_specs=[
            pl.BlockSpec((_BF,), lambda i: (i,)),
            pl.BlockSpec((_BF,), lambda i: (i,)),
        ],
        out_shape=[
            jax.ShapeDtypeStruct((nflat,), jnp.float32),
            jax.ShapeDtypeStruct((nflat,), jnp.float32),
        ],
    )(aggs1d, aggs1d, dn1d, sn1d, h01d)


def kernel(features, edge_index, W1, b1, W2, b2):
    src = edge_index[0]
    dst = edge_index[1]
    pad = EPAD - E
    src_gath = jnp.concatenate([src, jnp.zeros((pad,), jnp.int32)])
    src_deg = jnp.concatenate([src, jnp.full((pad,), N, jnp.int32)])
    dst_pad = jnp.concatenate([dst, jnp.full((pad,), N, jnp.int32)])
    src_gath = src_gath.reshape(NC * NS, NCHUNK, CHUNK)
    src_deg = src_deg.reshape(NC * NS, NCHUNK, CHUNK)
    dst_pad = dst_pad.reshape(NC * NS, NCHUNK, CHUNK)

    xpad = jnp.concatenate(
        [features, jnp.zeros((NPAD - N, D_IN), jnp.float32)])
    h0 = _mlp(xpad, W1, b1.reshape(1, -1), W2, b2.reshape(1, -1))
    norms = _sc_degrees(src_deg, dst_pad)
    sn1d = norms[0].reshape(-1)
    dn1d = norms[1].reshape(-1)
    h01d = h0.reshape(-1)
    hs1d = _scale(h01d, sn1d)

    def _step(carry, _):
        h1d, hs1d = carry
        aggs = _sc_step(hs1d.reshape(NPAD, N_CLASSES), src_gath, dst_pad)
        h1d, hs1d = _blend(aggs.reshape(-1), dn1d, sn1d, h01d)
        return (h1d, hs1d), None

    (h1d, _), _ = lax.scan(_step, (h01d, hs1d), None, length=K)
    return h1d.reshape(NPAD, N_CLASSES)[:N]


# speedup vs baseline: 2.5690x; 1.0450x over previous
import functools

import jax
import jax.numpy as jnp
from jax import lax
from jax.experimental import pallas as pl
from jax.experimental.pallas import tpu as pltpu
from jax.experimental.pallas import tpu_sc as plsc

N = 10000
E = 160000
D_IN = 256
H_FEATS = 512
N_CLASSES = 64
K = 10
ALPHA = 0.1

NC = 2
NS = 16
LANES = 16
NPAD = 10240
EPT = 5120
EPAD = NC * NS * EPT
CHUNK = 128
NCHUNK = EPT // CHUNK
NBUF = 4
NROUND = NCHUNK // NBUF
ROWS_PER_TILE = NPAD // NS
_MESH = plsc.VectorSubcoreMesh(core_axis_name="c", subcore_axis_name="s")
_SC_PARAMS = pltpu.CompilerParams(use_tc_tiling_on_sc=False)
_SC_PARAMS_NOLAYOUT = pltpu.CompilerParams(use_tc_tiling_on_sc=False,
                                           needs_layout_passes=False)

_BN = 1024
_BF = 2 * _BN * N_CLASSES


def _mlp_body(x_ref, w1_ref, b1_ref, w2_ref, b2_ref, o_ref):
    h = jnp.dot(x_ref[...], w1_ref[...], preferred_element_type=jnp.float32)
    h = jnp.maximum(h + b1_ref[...], 0.0)
    o_ref[...] = jnp.dot(h, w2_ref[...], preferred_element_type=jnp.float32) + b2_ref[...]


def _mlp(x, w1, b1, w2, b2):
    return pl.pallas_call(
        _mlp_body,
        grid=(NPAD // _BN,),
        in_specs=[
            pl.BlockSpec((_BN, D_IN), lambda i: (i, 0)),
            pl.BlockSpec((D_IN, H_FEATS), lambda i: (0, 0)),
            pl.BlockSpec((1, H_FEATS), lambda i: (0, 0)),
            pl.BlockSpec((H_FEATS, N_CLASSES), lambda i: (0, 0)),
            pl.BlockSpec((1, N_CLASSES), lambda i: (0, 0)),
        ],
        out_specs=pl.BlockSpec((_BN, N_CLASSES), lambda i: (i, 0)),
        out_shape=jax.ShapeDtypeStruct((NPAD, N_CLASSES), jnp.float32),
    )(x, w1, b1, w2, b2)


def _rsqrt16(x):
    i = plsc.bitcast(x, jnp.int32)
    i = jnp.full((LANES,), 0x5F3759DF, jnp.int32) - lax.shift_right_logical(i, 1)
    y = plsc.bitcast(i, jnp.float32)
    for _ in range(3):
        y = y * (1.5 - 0.5 * x * y * y)
    return y


@functools.partial(
    pl.kernel,
    out_type=jax.ShapeDtypeStruct((2, NPAD, N_CLASSES), jnp.float32),
    mesh=_MESH,
    compiler_params=_SC_PARAMS_NOLAYOUT,
    scratch_types=[
        pltpu.VMEM((NCHUNK, CHUNK), jnp.int32),
        pltpu.VMEM((NPAD,), jnp.float32),
        pltpu.VMEM((NPAD,), jnp.float32),
        pltpu.VMEM((NS, ROWS_PER_TILE), jnp.float32),
        pltpu.VMEM((ROWS_PER_TILE,), jnp.float32),
        pltpu.VMEM((ROWS_PER_TILE, N_CLASSES), jnp.float32),
        pltpu.VMEM_SHARED((NS, NPAD), jnp.float32),
        pltpu.VMEM_SHARED((NS, NPAD), jnp.float32),
        pltpu.SemaphoreType.DMA,
    ],
)
def _sc_degrees(src_hbm, dst_hbm, out_hbm, idx_v, hsrc_v, hdst_v,
                acc_v, nrm_v, exp_v, stage_src_sh, stage_dst_sh, sem_a):
    cid = lax.axis_index("c")
    sid = lax.axis_index("s")

    @pl.when(cid == 0)
    def _():
        @pl.loop(0, NPAD, step=LANES)
        def _(i):
            hsrc_v[pl.ds(i, LANES)] = jnp.zeros((LANES,), jnp.float32)
            hdst_v[pl.ds(i, LANES)] = jnp.zeros((LANES,), jnp.float32)

        ones16 = jnp.full((LANES,), 1.0, jnp.float32)

        for part in range(2):
            for which, ehbm, hist in ((0, src_hbm, hsrc_v), (1, dst_hbm, hdst_v)):
                pltpu.sync_copy(ehbm.at[part * NS + sid], idx_v)

                @pl.loop(0, NCHUNK)
                def _(c):
                    @pl.loop(0, CHUNK, step=LANES)
                    def _(j):
                        plsc.addupdate_scatter(hist, [idx_v[c, pl.ds(j, LANES)]],
                                               ones16)

        pltpu.sync_copy(hsrc_v, stage_src_sh.at[sid])
        pltpu.sync_copy(hdst_v, stage_dst_sh.at[sid])
        plsc.subcore_barrier()

        base = sid * ROWS_PER_TILE
        for which, stage in ((0, stage_src_sh), (1, stage_dst_sh)):
            for t in range(NS):
                pltpu.async_copy(stage.at[t, pl.ds(base, ROWS_PER_TILE)],
                                 acc_v.at[t], sem_a)
            for t in range(NS):
                pltpu.make_async_copy(stage.at[t, pl.ds(base, ROWS_PER_TILE)],
                                      acc_v.at[t], sem_a).wait()

            @pl.loop(0, ROWS_PER_TILE, step=LANES)
            def _(j):
                s = acc_v[0, pl.ds(j, LANES)]
                for t in range(1, NS):
                    s = s + acc_v[t, pl.ds(j, LANES)]
                nrm_v[pl.ds(j, LANES)] = _rsqrt16(jnp.maximum(s, 1.0))

            @pl.loop(0, ROWS_PER_TILE)
            def _(r):
                v = plsc.load_gather(nrm_v, [jnp.full((LANES,), r, jnp.int32)])
                @pl.loop(0, N_CLASSES, step=LANES)
                def _(jj):
                    exp_v[r, pl.ds(jj, LANES)] = v

            pltpu.sync_copy(exp_v, out_hbm.at[which, pl.ds(base, ROWS_PER_TILE)])


@functools.partial(
    pl.kernel,
    out_type=jax.ShapeDtypeStruct((NC, NPAD, N_CLASSES), jnp.float32),
    mesh=_MESH,
    compiler_params=_SC_PARAMS,
    scratch_types=[
        pltpu.VMEM((NCHUNK, CHUNK), jnp.int32),
        pltpu.VMEM((NCHUNK, CHUNK), jnp.int32),
        pltpu.VMEM((NBUF, CHUNK, N_CLASSES), jnp.float32),
        pltpu.VMEM_SHARED((NPAD, N_CLASSES), jnp.float32),
        pltpu.VMEM_SHARED((NPAD, N_CLASSES), jnp.float32),
        pltpu.SemaphoreType.DMA((NBUF,)),
        pltpu.SemaphoreType.DMA((NBUF,)),
        pltpu.SemaphoreType.DMA,
    ],
)
def _sc_step(hs_hbm, src_hbm, dst_hbm, out_hbm, sidx_v, didx_v, rows_v,
             hs_sh, agg_sh, gsem, ssem, sem_z):
    cid = lax.axis_index("c")
    sid = lax.axis_index("s")
    wid = cid * NS + sid

    pltpu.async_copy(src_hbm.at[wid], sidx_v, sem_z)
    pltpu.async_copy(dst_hbm.at[wid], didx_v, sem_z)
    hrows = NPAD // NS
    pltpu.async_copy(hs_hbm.at[pl.ds(sid * hrows, hrows)],
                     hs_sh.at[pl.ds(sid * hrows, hrows)], sem_z)

    zero_v = rows_v.at[0]

    @pl.loop(0, CHUNK)
    def _(r):
        @pl.loop(0, N_CLASSES, step=LANES)
        def _(j):
            zero_v[r, pl.ds(j, LANES)] = jnp.zeros((LANES,), jnp.float32)

    base = sid * ROWS_PER_TILE

    @pl.loop(0, ROWS_PER_TILE // CHUNK)
    def _(b):
        pltpu.sync_copy(zero_v, agg_sh.at[pl.ds(base + b * CHUNK, CHUNK)])

    pltpu.make_async_copy(src_hbm.at[wid], sidx_v, sem_z).wait()
    pltpu.make_async_copy(dst_hbm.at[wid], didx_v, sem_z).wait()
    pltpu.make_async_copy(hs_hbm.at[pl.ds(sid * hrows, hrows)],
                          hs_sh.at[pl.ds(sid * hrows, hrows)], sem_z).wait()
    plsc.subcore_barrier()

    def _gather(c, b):
        pltpu.async_copy(hs_sh.at[sidx_v.at[c]], rows_v.at[b], gsem.at[b])

    def _gather_wait(c, b):
        pltpu.make_async_copy(hs_sh.at[sidx_v.at[c]], rows_v.at[b], gsem.at[b]).wait()

    def _scatter(c, b):
        pltpu.async_copy(rows_v.at[b], agg_sh.at[didx_v.at[c]], ssem.at[b], add=True)

    def _scatter_wait(c, b):
        pltpu.make_async_copy(rows_v.at[b], agg_sh.at[didx_v.at[c]], ssem.at[b]).wait()

    for b in range(NBUF):
        _gather(b, b)

    @pl.loop(0, NROUND - 1)
    def _(r):
        c0 = r * NBUF
        for b in range(NBUF):
            _gather_wait(c0 + b, b)
            _scatter(c0 + b, b)
        for b in range(NBUF):
            _scatter_wait(c0 + b, b)
            _gather(c0 + NBUF + b, b)

    c0 = (NROUND - 1) * NBUF
    for b in range(NBUF):
        _gather_wait(c0 + b, b)
        _scatter(c0 + b, b)
    for b in range(NBUF):
        _scatter_wait(c0 + b, b)

    plsc.subcore_barrier()
    pltpu.sync_copy(agg_sh.at[pl.ds(base, ROWS_PER_TILE)],
                    out_hbm.at[cid, pl.ds(base, ROWS_PER_TILE)])


def _scale_body(a_ref, b_ref, o_ref):
    o_ref[...] = a_ref[...] * b_ref[...]


def _scale(a1d, b1d):
    nflat = NPAD * N_CLASSES
    return pl.pallas_call(
        _scale_body,
        grid=(nflat // _BF,),
        in_specs=[
            pl.BlockSpec((_BF,), lambda i: (i,)),
            pl.BlockSpec((_BF,), lambda i: (i,)),
        ],
        out_specs=pl.BlockSpec((_BF,), lambda i: (i,)),
        out_shape=jax.ShapeDtypeStruct((nflat,), jnp.float32),
    )(a1d, b1d)


def _blend_body(a0_ref, a1_ref, dn_ref, sn_ref, h0_ref, h_ref, hs_ref):
    h = ((1.0 - ALPHA) * (a0_ref[...] + a1_ref[...]) * dn_ref[...]
         + ALPHA * h0_ref[...])
    h_ref[...] = h
    hs_ref[...] = h * sn_ref[...]


def _blend(aggs1d, dn1d, sn1d, h01d):
    nflat = NPAD * N_CLASSES
    return pl.pallas_call(
        _blend_body,
        grid=(nflat // _BF,),
        in_specs=[
            pl.BlockSpec((_BF,), lambda i: (i,)),
            pl.BlockSpec((_BF,), lambda i: (i + NPAD * N_CLASSES // _BF,)),
            pl.BlockSpec((_BF,), lambda i: (i,)),
            pl.BlockSpec((_BF,), lambda i: (i,)),
            pl.BlockSpec((_BF,), lambda i: (i,)),
        ],
        out_specs=[
            pl.BlockSpec((_BF,), lambda i: (i,)),
            pl.BlockSpec((_BF,), lambda i: (i,)),
        ],
        out_shape=[
            jax.ShapeDtypeStruct((nflat,), jnp.float32),
            jax.ShapeDtypeStruct((nflat,), jnp.float32),
        ],
    )(aggs1d, aggs1d, dn1d, sn1d, h01d)


def kernel(features, edge_index, W1, b1, W2, b2):
    src = edge_index[0]
    dst = edge_index[1]
    pad = EPAD - E
    src_gath = jnp.concatenate([src, jnp.zeros((pad,), jnp.int32)])
    src_deg = jnp.concatenate([src, jnp.full((pad,), N, jnp.int32)])
    dst_pad = jnp.concatenate([dst, jnp.full((pad,), N, jnp.int32)])
    src_gath = src_gath.reshape(NC * NS, NCHUNK, CHUNK)
    src_deg = src_deg.reshape(NC * NS, NCHUNK, CHUNK)
    dst_pad = dst_pad.reshape(NC * NS, NCHUNK, CHUNK)

    xpad = jnp.concatenate(
        [features, jnp.zeros((NPAD - N, D_IN), jnp.float32)])
    h0 = _mlp(xpad, W1, b1.reshape(1, -1), W2, b2.reshape(1, -1))
    norms = _sc_degrees(src_deg, dst_pad)
    sn1d = norms[0].reshape(-1)
    dn1d = norms[1].reshape(-1)
    h01d = h0.reshape(-1)
    hs1d = _scale(h01d, sn1d)

    h1d = h01d
    for _ in range(K):
        aggs = _sc_step(hs1d.reshape(NPAD, N_CLASSES), src_gath, dst_pad)
        h1d, hs1d = _blend(aggs.reshape(-1), dn1d, sn1d, h01d)
    return h1d.reshape(NPAD, N_CLASSES)[:N]


# speedup vs baseline: 2.6169x; 1.0186x over previous
import functools

import jax
import jax.numpy as jnp
from jax import lax
from jax.experimental import pallas as pl
from jax.experimental.pallas import tpu as pltpu
from jax.experimental.pallas import tpu_sc as plsc

N = 10000
E = 160000
D_IN = 256
H_FEATS = 512
N_CLASSES = 64
K = 10
ALPHA = 0.1

NC = 2
NS = 16
LANES = 16
NPAD = 10240
EPT = 5120
EPAD = NC * NS * EPT
CHUNK = 128
NCHUNK = EPT // CHUNK
NBUF = 4
NROUND = NCHUNK // NBUF
ROWS_PER_TILE = NPAD // NS
_MESH = plsc.VectorSubcoreMesh(core_axis_name="c", subcore_axis_name="s")
_SC_PARAMS = pltpu.CompilerParams(use_tc_tiling_on_sc=False)
_SC_PARAMS_NOLAYOUT = pltpu.CompilerParams(use_tc_tiling_on_sc=False,
                                           needs_layout_passes=False)

_BN = 1024
_BF = 5 * _BN * N_CLASSES


def _mlp_body(x_ref, w1_ref, b1_ref, w2_ref, b2_ref, o_ref):
    h = jnp.dot(x_ref[...], w1_ref[...], preferred_element_type=jnp.float32)
    h = jnp.maximum(h + b1_ref[...], 0.0)
    o_ref[...] = jnp.dot(h, w2_ref[...], preferred_element_type=jnp.float32) + b2_ref[...]


def _mlp(x, w1, b1, w2, b2):
    return pl.pallas_call(
        _mlp_body,
        grid=(NPAD // _BN,),
        in_specs=[
            pl.BlockSpec((_BN, D_IN), lambda i: (i, 0)),
            pl.BlockSpec((D_IN, H_FEATS), lambda i: (0, 0)),
            pl.BlockSpec((1, H_FEATS), lambda i: (0, 0)),
            pl.BlockSpec((H_FEATS, N_CLASSES), lambda i: (0, 0)),
            pl.BlockSpec((1, N_CLASSES), lambda i: (0, 0)),
        ],
        out_specs=pl.BlockSpec((_BN, N_CLASSES), lambda i: (i, 0)),
        out_shape=jax.ShapeDtypeStruct((NPAD, N_CLASSES), jnp.float32),
    )(x, w1, b1, w2, b2)


def _rsqrt16(x):
    i = plsc.bitcast(x, jnp.int32)
    i = jnp.full((LANES,), 0x5F3759DF, jnp.int32) - lax.shift_right_logical(i, 1)
    y = plsc.bitcast(i, jnp.float32)
    for _ in range(3):
        y = y * (1.5 - 0.5 * x * y * y)
    return y


@functools.partial(
    pl.kernel,
    out_type=jax.ShapeDtypeStruct((2, NPAD, N_CLASSES), jnp.float32),
    mesh=_MESH,
    compiler_params=_SC_PARAMS_NOLAYOUT,
    scratch_types=[
        pltpu.VMEM((NCHUNK, CHUNK), jnp.int32),
        pltpu.VMEM((NPAD,), jnp.float32),
        pltpu.VMEM((NPAD,), jnp.float32),
        pltpu.VMEM((NS, ROWS_PER_TILE), jnp.float32),
        pltpu.VMEM((ROWS_PER_TILE,), jnp.float32),
        pltpu.VMEM((ROWS_PER_TILE, N_CLASSES), jnp.float32),
        pltpu.VMEM_SHARED((NS, NPAD), jnp.float32),
        pltpu.VMEM_SHARED((NS, NPAD), jnp.float32),
        pltpu.SemaphoreType.DMA,
    ],
)
def _sc_degrees(src_hbm, dst_hbm, out_hbm, idx_v, hsrc_v, hdst_v,
                acc_v, nrm_v, exp_v, stage_src_sh, stage_dst_sh, sem_a):
    cid = lax.axis_index("c")
    sid = lax.axis_index("s")

    @pl.when(cid == 0)
    def _():
        @pl.loop(0, NPAD, step=LANES)
        def _(i):
            hsrc_v[pl.ds(i, LANES)] = jnp.zeros((LANES,), jnp.float32)
            hdst_v[pl.ds(i, LANES)] = jnp.zeros((LANES,), jnp.float32)

        ones16 = jnp.full((LANES,), 1.0, jnp.float32)

        for part in range(2):
            for which, ehbm, hist in ((0, src_hbm, hsrc_v), (1, dst_hbm, hdst_v)):
                pltpu.sync_copy(ehbm.at[part * NS + sid], idx_v)

                @pl.loop(0, NCHUNK)
                def _(c):
                    @pl.loop(0, CHUNK, step=LANES)
                    def _(j):
                        plsc.addupdate_scatter(hist, [idx_v[c, pl.ds(j, LANES)]],
                                               ones16)

        pltpu.sync_copy(hsrc_v, stage_src_sh.at[sid])
        pltpu.sync_copy(hdst_v, stage_dst_sh.at[sid])
        plsc.subcore_barrier()

        base = sid * ROWS_PER_TILE
        for which, stage in ((0, stage_src_sh), (1, stage_dst_sh)):
            for t in range(NS):
                pltpu.async_copy(stage.at[t, pl.ds(base, ROWS_PER_TILE)],
                                 acc_v.at[t], sem_a)
            for t in range(NS):
                pltpu.make_async_copy(stage.at[t, pl.ds(base, ROWS_PER_TILE)],
                                      acc_v.at[t], sem_a).wait()

            @pl.loop(0, ROWS_PER_TILE, step=LANES)
            def _(j):
                s = acc_v[0, pl.ds(j, LANES)]
                for t in range(1, NS):
                    s = s + acc_v[t, pl.ds(j, LANES)]
                nrm_v[pl.ds(j, LANES)] = _rsqrt16(jnp.maximum(s, 1.0))

            @pl.loop(0, ROWS_PER_TILE)
            def _(r):
                v = plsc.load_gather(nrm_v, [jnp.full((LANES,), r, jnp.int32)])
                @pl.loop(0, N_CLASSES, step=LANES)
                def _(jj):
                    exp_v[r, pl.ds(jj, LANES)] = v

            pltpu.sync_copy(exp_v, out_hbm.at[which, pl.ds(base, ROWS_PER_TILE)])


@functools.partial(
    pl.kernel,
    out_type=jax.ShapeDtypeStruct((NC, NPAD, N_CLASSES), jnp.float32),
    mesh=_MESH,
    compiler_params=_SC_PARAMS,
    scratch_types=[
        pltpu.VMEM((NCHUNK, CHUNK), jnp.int32),
        pltpu.VMEM((NCHUNK, CHUNK), jnp.int32),
        pltpu.VMEM((NBUF, CHUNK, N_CLASSES), jnp.float32),
        pltpu.VMEM_SHARED((NPAD, N_CLASSES), jnp.float32),
        pltpu.VMEM_SHARED((NPAD, N_CLASSES), jnp.float32),
        pltpu.SemaphoreType.DMA((NBUF,)),
        pltpu.SemaphoreType.DMA((NBUF,)),
        pltpu.SemaphoreType.DMA,
    ],
)
def _sc_step(hs_hbm, src_hbm, dst_hbm, out_hbm, sidx_v, didx_v, rows_v,
             hs_sh, agg_sh, gsem, ssem, sem_z):
    cid = lax.axis_index("c")
    sid = lax.axis_index("s")
    wid = cid * NS + sid

    pltpu.async_copy(src_hbm.at[wid], sidx_v, sem_z)
    pltpu.async_copy(dst_hbm.at[wid], didx_v, sem_z)
    hrows = NPAD // NS
    pltpu.async_copy(hs_hbm.at[pl.ds(sid * hrows, hrows)],
                     hs_sh.at[pl.ds(sid * hrows, hrows)], sem_z)

    zero_v = rows_v.at[0]

    @pl.loop(0, CHUNK)
    def _(r):
        @pl.loop(0, N_CLASSES, step=LANES)
        def _(j):
            zero_v[r, pl.ds(j, LANES)] = jnp.zeros((LANES,), jnp.float32)

    base = sid * ROWS_PER_TILE

    @pl.loop(0, ROWS_PER_TILE // CHUNK)
    def _(b):
        pltpu.sync_copy(zero_v, agg_sh.at[pl.ds(base + b * CHUNK, CHUNK)])

    pltpu.make_async_copy(src_hbm.at[wid], sidx_v, sem_z).wait()
    pltpu.make_async_copy(dst_hbm.at[wid], didx_v, sem_z).wait()
    pltpu.make_async_copy(hs_hbm.at[pl.ds(sid * hrows, hrows)],
                          hs_sh.at[pl.ds(sid * hrows, hrows)], sem_z).wait()
    plsc.subcore_barrier()

    def _gather(c, b):
        pltpu.async_copy(hs_sh.at[sidx_v.at[c]], rows_v.at[b], gsem.at[b])

    def _gather_wait(c, b):
        pltpu.make_async_copy(hs_sh.at[sidx_v.at[c]], rows_v.at[b], gsem.at[b]).wait()

    def _scatter(c, b):
        pltpu.async_copy(rows_v.at[b], agg_sh.at[didx_v.at[c]], ssem.at[b], add=True)

    def _scatter_wait(c, b):
        pltpu.make_async_copy(rows_v.at[b], agg_sh.at[didx_v.at[c]], ssem.at[b]).wait()

    for b in range(NBUF):
        _gather(b, b)

    @pl.loop(0, NROUND - 1)
    def _(r):
        c0 = r * NBUF
        for b in range(NBUF):
            _gather_wait(c0 + b, b)
            _scatter(c0 + b, b)
        for b in range(NBUF):
            _scatter_wait(c0 + b, b)
            _gather(c0 + NBUF + b, b)

    c0 = (NROUND - 1) * NBUF
    for b in range(NBUF):
        _gather_wait(c0 + b, b)
        _scatter(c0 + b, b)
    for b in range(NBUF):
        _scatter_wait(c0 + b, b)

    plsc.subcore_barrier()
    pltpu.sync_copy(agg_sh.at[pl.ds(base, ROWS_PER_TILE)],
                    out_hbm.at[cid, pl.ds(base, ROWS_PER_TILE)])


def _scale_body(a_ref, b_ref, o_ref):
    o_ref[...] = a_ref[...] * b_ref[...]


def _scale(a1d, b1d):
    nflat = NPAD * N_CLASSES
    return pl.pallas_call(
        _scale_body,
        grid=(nflat // _BF,),
        in_specs=[
            pl.BlockSpec((_BF,), lambda i: (i,)),
            pl.BlockSpec((_BF,), lambda i: (i,)),
        ],
        out_specs=pl.BlockSpec((_BF,), lambda i: (i,)),
        out_shape=jax.ShapeDtypeStruct((nflat,), jnp.float32),
    )(a1d, b1d)


def _blend_body(a0_ref, a1_ref, dn_ref, sn_ref, h0_ref, h_ref, hs_ref):
    h = ((1.0 - ALPHA) * (a0_ref[...] + a1_ref[...]) * dn_ref[...]
         + ALPHA * h0_ref[...])
    h_ref[...] = h
    hs_ref[...] = h * sn_ref[...]


def _blend(aggs1d, dn1d, sn1d, h01d):
    nflat = NPAD * N_CLASSES
    return pl.pallas_call(
        _blend_body,
        grid=(nflat // _BF,),
        in_specs=[
            pl.BlockSpec((_BF,), lambda i: (i,)),
            pl.BlockSpec((_BF,), lambda i: (i + NPAD * N_CLASSES // _BF,)),
            pl.BlockSpec((_BF,), lambda i: (i,)),
            pl.BlockSpec((_BF,), lambda i: (i,)),
            pl.BlockSpec((_BF,), lambda i: (i,)),
        ],
        out_specs=[
            pl.BlockSpec((_BF,), lambda i: (i,)),
            pl.BlockSpec((_BF,), lambda i: (i,)),
        ],
        out_shape=[
            jax.ShapeDtypeStruct((nflat,), jnp.float32),
            jax.ShapeDtypeStruct((nflat,), jnp.float32),
        ],
    )(aggs1d, aggs1d, dn1d, sn1d, h01d)


def kernel(features, edge_index, W1, b1, W2, b2):
    src = edge_index[0]
    dst = edge_index[1]
    pad = EPAD - E
    src_gath = jnp.concatenate([src, jnp.zeros((pad,), jnp.int32)])
    src_deg = jnp.concatenate([src, jnp.full((pad,), N, jnp.int32)])
    dst_pad = jnp.concatenate([dst, jnp.full((pad,), N, jnp.int32)])
    src_gath = src_gath.reshape(NC * NS, NCHUNK, CHUNK)
    src_deg = src_deg.reshape(NC * NS, NCHUNK, CHUNK)
    dst_pad = dst_pad.reshape(NC * NS, NCHUNK, CHUNK)

    xpad = jnp.concatenate(
        [features, jnp.zeros((NPAD - N, D_IN), jnp.float32)])
    h0 = _mlp(xpad, W1, b1.reshape(1, -1), W2, b2.reshape(1, -1))
    norms = _sc_degrees(src_deg, dst_pad)
    sn1d = norms[0].reshape(-1)
    dn1d = norms[1].reshape(-1)
    h01d = h0.reshape(-1)
    hs1d = _scale(h01d, sn1d)

    h1d = h01d
    for _ in range(K):
        aggs = _sc_step(hs1d.reshape(NPAD, N_CLASSES), src_gath, dst_pad)
        h1d, hs1d = _blend(aggs.reshape(-1), dn1d, sn1d, h01d)
    return h1d.reshape(NPAD, N_CLASSES)[:N]
